# baseline (speedup 1.0000x reference)
import jax
import jax.numpy as jnp
from jax.experimental import pallas as pl

BATCH = 1024
FEATS = 26
DEPTH = 1000
BLOCK_B = 128


def _onehot_block(idx_ref, out_ref):
    idx = idx_ref[...]
    col = jax.lax.broadcasted_iota(jnp.int32, (BLOCK_B, FEATS, DEPTH), 2)
    out_ref[...] = (col == idx[:, :, None]).astype(jnp.float32)


def kernel(indices):
    return pl.pallas_call(
        _onehot_block,
        grid=(BATCH // BLOCK_B,),
        in_specs=[pl.BlockSpec((BLOCK_B, FEATS), lambda i: (i, 0))],
        out_specs=pl.BlockSpec((BLOCK_B, FEATS, DEPTH), lambda i: (i, 0, 0)),
        out_shape=jax.ShapeDtypeStruct((BATCH, FEATS, DEPTH), jnp.float32),
    )(indices)


# speedup vs baseline: 4.7162x; 4.7162x over previous
import jax
import jax.numpy as jnp
from jax.experimental import pallas as pl

DEPTH = 1000
BATCH = 1024
FEATS = 26
BLOCK_F = 2


def _onehot_t_block(idx_ref, out_ref):
    idx = idx_ref[...]
    k = jax.lax.broadcasted_iota(jnp.int32, (BLOCK_F, DEPTH, BATCH), 1)
    out_ref[...] = (k == idx).astype(jnp.float32)


def kernel(indices):
    idx_t = indices.T.reshape(FEATS, 1, BATCH)
    out_t = pl.pallas_call(
        _onehot_t_block,
        grid=(FEATS // BLOCK_F,),
        in_specs=[pl.BlockSpec((BLOCK_F, 1, BATCH), lambda i: (i, 0, 0))],
        out_specs=pl.BlockSpec((BLOCK_F, DEPTH, BATCH), lambda i: (i, 0, 0)),
        out_shape=jax.ShapeDtypeStruct((FEATS, DEPTH, BATCH), jnp.float32),
    )(idx_t)
    return jnp.transpose(out_t, (2, 0, 1))


# speedup vs baseline: 4.8102x; 1.0199x over previous
import jax
import jax.numpy as jnp
from jax.experimental import pallas as pl

DEPTH = 1000
BATCH = 1024
FEATS = 26
BLOCK_F = 1


def _onehot_t_block(idx_ref, out_ref):
    idx = idx_ref[...]
    k = jax.lax.broadcasted_iota(jnp.int32, (BLOCK_F, DEPTH, BATCH), 1)
    out_ref[...] = (k == idx).astype(jnp.float32)


def kernel(indices):
    idx_t = indices.T.reshape(FEATS, 1, BATCH)
    out_t = pl.pallas_call(
        _onehot_t_block,
        grid=(FEATS // BLOCK_F,),
        in_specs=[pl.BlockSpec((BLOCK_F, 1, BATCH), lambda i: (i, 0, 0))],
        out_specs=pl.BlockSpec((BLOCK_F, DEPTH, BATCH), lambda i: (i, 0, 0)),
        out_shape=jax.ShapeDtypeStruct((FEATS, DEPTH, BATCH), jnp.float32),
    )(idx_t)
    return jnp.transpose(out_t, (2, 0, 1))
